# Initial kernel scaffold; baseline (speedup 1.0000x reference)
import jax
import jax.numpy as jnp
from jax.experimental import pallas as pl


def kernel(node_ids, messages, timestamps, memory, last_update, W_i, W_h, b_i, b_h):
    raise NotImplementedError("write your pallas kernel here")


# speedup vs baseline: 1.3006x; 1.3006x over previous
import functools

import jax
import jax.numpy as jnp
from jax import lax
from jax.experimental import pallas as pl
from jax.experimental.pallas import tpu as pltpu
from jax.experimental.pallas import tpu_sc as plsc

B = 50000
N = 100000
D = 128
G3 = 3 * D

NC = 2
NS = 16
NW = NC * NS
L = 16

SUB = 112
NSUB = 14
CH = SUB * NSUB
B_PAD = CH * NW
NVREG = B // L

NSHARD = 3200
N_PAD = NSHARD * NW

BB = 400


def _worker_id():
    return lax.axis_index("s") * NC + lax.axis_index("c")


def _k1_body(ids_hbm, ts_hbm, mem_hbm, lu_hbm,
             w_hbm, h_hbm, dt_hbm,
             ids_v, w_v, rows_v, tprev_v, ts_v, dt_v, sem):
    wid = _worker_id()

    pltpu.sync_copy(ids_hbm, ids_v)

    lo = wid * NSHARD
    lanes = lax.iota(jnp.int32, L)

    shift_idx = jnp.minimum(lanes + 1, L - 1)

    def scan_step(i, carry):
        idv = ids_v[pl.ds(i * L, L)]
        bv = lanes + i * L
        key = (idv << 4) | lanes
        skey, sb = plsc.sort_key_val(key, bv)
        sid = skey >> 4
        nid = sid.at[shift_idx].get(mode="promise_in_bounds")
        last = (sid != nid) | (lanes == L - 1)
        inr = (sid >= lo) & (sid < lo + NSHARD)
        loc = jnp.where(inr, sid - lo, 0)
        plsc.store_scatter(w_v, [loc], sb, mask=last & inr)
        return carry

    lax.fori_loop(0, NVREG, scan_step, 0)
    pltpu.sync_copy(w_v, w_hbm.at[pl.ds(lo, NSHARD)])

    base = wid * CH

    def gather_step(j, carry):
        off = base + j * SUB
        idx = ids_v.at[pl.ds(off, SUB)]
        pltpu.async_copy(mem_hbm.at[idx], rows_v, sem).wait()
        pltpu.sync_copy(rows_v, h_hbm.at[pl.ds(off, SUB)])
        pltpu.async_copy(lu_hbm.at[idx], tprev_v, sem).wait()
        pltpu.sync_copy(ts_hbm.at[wid, j], ts_v)
        for q in range(SUB // L):
            s = pl.ds(q * L, L)
            dt_v[s] = ts_v[s] - tprev_v[s]
        pltpu.sync_copy(dt_v, dt_hbm.at[wid, j])
        return carry

    lax.fori_loop(0, NSUB, gather_step, 0)


_k1 = functools.partial(
    pl.kernel,
    out_type=(
        jax.ShapeDtypeStruct((N_PAD,), jnp.int32),
        jax.ShapeDtypeStruct((B_PAD, D), jnp.float32),
        jax.ShapeDtypeStruct((NW, NSUB, SUB), jnp.float32),
    ),
    mesh=plsc.VectorSubcoreMesh(
        core_axis_name="c", subcore_axis_name="s", num_cores=NC,
        num_subcores=NS),
    scratch_types=[
        pltpu.VMEM((B_PAD,), jnp.int32),
        pltpu.VMEM((NSHARD,), jnp.int32),
        pltpu.VMEM((SUB, D), jnp.float32),
        pltpu.VMEM((SUB,), jnp.float32),
        pltpu.VMEM((SUB,), jnp.float32),
        pltpu.VMEM((SUB,), jnp.float32),
        pltpu.SemaphoreType.DMA,
    ],
    compiler_params=pltpu.CompilerParams(needs_layout_passes=False),
)(_k1_body)


def _gru_body(msg_ref, h_ref, dt_ref, wm_ref, wh_ref, wdt_ref, bi_ref,
              bh_ref, out_ref):
    msg = msg_ref[...]
    h = h_ref[...]
    dt = dt_ref[...]
    gx = (jnp.dot(msg, wm_ref[...], preferred_element_type=jnp.float32)
          + dt * wdt_ref[...] + bi_ref[...])
    gh = (jnp.dot(h, wh_ref[...], preferred_element_type=jnp.float32)
          + bh_ref[...])
    r = jax.nn.sigmoid(gx[:, :D] + gh[:, :D])
    z = jax.nn.sigmoid(gx[:, D:2 * D] + gh[:, D:2 * D])
    n = jnp.tanh(gx[:, 2 * D:] + r * gh[:, 2 * D:])
    out_ref[...] = (1.0 - z) * n + z * h


_gru = pl.pallas_call(
    _gru_body,
    grid=(B // BB,),
    in_specs=[
        pl.BlockSpec((BB, D), lambda i: (i, 0)),
        pl.BlockSpec((BB, D), lambda i: (i, 0)),
        pl.BlockSpec((BB, 1), lambda i: (i, 0)),
        pl.BlockSpec((D, G3), lambda i: (0, 0)),
        pl.BlockSpec((D, G3), lambda i: (0, 0)),
        pl.BlockSpec((1, G3), lambda i: (0, 0)),
        pl.BlockSpec((1, G3), lambda i: (0, 0)),
        pl.BlockSpec((1, G3), lambda i: (0, 0)),
    ],
    out_specs=pl.BlockSpec((BB, D), lambda i: (i, 0)),
    out_shape=jax.ShapeDtypeStruct((B, D), jnp.float32),
)


def _k3_body(ids_hbm, w_hbm, hnew_hbm, mem_hbm, ids_v, src_v, rows_v, sem):
    wid = _worker_id()
    pltpu.sync_copy(ids_hbm.at[wid], ids_v)

    def step(j, carry):
        pltpu.async_copy(w_hbm.at[ids_v.at[j]], src_v.at[j], sem).wait()
        pltpu.async_copy(hnew_hbm.at[src_v.at[j]], rows_v, sem).wait()
        pltpu.async_copy(rows_v, mem_hbm.at[ids_v.at[j]], sem).wait()
        return carry

    lax.fori_loop(0, NSUB, step, 0)


_k3 = functools.partial(
    pl.kernel,
    mesh=plsc.VectorSubcoreMesh(
        core_axis_name="c", subcore_axis_name="s", num_cores=NC,
        num_subcores=NS),
    scratch_types=[
        pltpu.VMEM((NSUB, SUB), jnp.int32),
        pltpu.VMEM((NSUB, SUB), jnp.int32),
        pltpu.VMEM((SUB, D), jnp.float32),
        pltpu.SemaphoreType.DMA,
    ],
)(_k3_body)


def kernel(node_ids, messages, timestamps, memory, last_update, W_i, W_h,
           b_i, b_h):
    ids = node_ids.astype(jnp.int32)
    pad = B_PAD - B
    ids_pad = jnp.concatenate([ids, jnp.broadcast_to(ids[-1], (pad,))])
    ts_pad = jnp.concatenate(
        [timestamps.astype(jnp.float32), jnp.zeros((pad,), jnp.float32)])
    ts_resh = ts_pad.reshape(NW, NSUB, SUB)

    w_tab, h_gat, dt = _k1(ids_pad, ts_resh, memory, last_update)
    dt_col = dt.reshape(B_PAD, 1)

    wm = W_i[:, :D].T
    wh = W_h.T
    wdt = W_i[:, D].reshape(1, G3)
    bi = b_i.reshape(1, G3)
    bh = b_h.reshape(1, G3)

    h_new = _gru(messages, h_gat, dt_col, wm, wh, wdt, bi, bh)

    out_ref = jax.new_ref(memory)
    _k3(ids_pad.reshape(NW, NSUB, SUB), w_tab, h_new, out_ref)
    return out_ref[...]
